# baseline (speedup 1.0000x reference)
import jax
import jax.numpy as jnp
from jax.experimental import pallas as pl
from jax.experimental.pallas import tpu as pltpu

_K = 32
_H = 128
_N_TYPES = 4
_BLOCK = 400


def _tree_cell_kernel(x_ref, nh_ref, nc_ref, tf_ref,
                      G_ref, Uf_ref, bias_ref,
                      h_ref, c_ref):
    B = x_ref.shape[0]
    x = x_ref[:]
    nh = nh_ref[:]
    nc = nc_ref[:]
    tf = tf_ref[:]

    nhb = nh.astype(jnp.bfloat16)
    h_tilde = jnp.sum(nh, axis=1)
    xh = jnp.concatenate([x, h_tilde], axis=1).astype(jnp.bfloat16)
    nhr = nhb.reshape(B * _K, _H)

    def type_pre(t):
        P = jnp.dot(xh, G_ref[t], preferred_element_type=jnp.float32)
        P = P + bias_ref[t]
        fp = jnp.dot(nhr, Uf_ref[t],
                     preferred_element_type=jnp.float32).reshape(B, _K, _H)
        return P, fp

    P, fp = type_pre(0)
    for t in range(1, _N_TYPES):
        m = tf == t
        P_t, fp_t = type_pre(t)
        P = jnp.where(jnp.concatenate([m, m, m, m], axis=1), P_t, P)
        fp = jnp.where(m[:, None, :], fp_t, fp)
    fp = fp + P[:, 3 * _H:][:, None, :]

    pi = P[:, :_H]
    po = P[:, _H:2 * _H]
    pu = P[:, 2 * _H:3 * _H]
    f_gate = jax.nn.sigmoid(fp)
    c_aggr = jnp.sum(f_gate * nc, axis=1)
    c = jax.nn.sigmoid(pi) * jnp.tanh(pu) + c_aggr
    h = jax.nn.sigmoid(po) * jnp.tanh(c)
    h_ref[:] = h
    c_ref[:] = c


@jax.jit
def kernel(x, n_h, n_c, type_id, W_iou, b_iou, U_iou, W_f, U_f, b_f):
    n = x.shape[0]
    H = _H
    T = _N_TYPES
    top = jnp.concatenate([W_iou, W_f], axis=2)
    bot = jnp.concatenate([U_iou, jnp.zeros((T, H, H), W_iou.dtype)], axis=2)
    G = jnp.concatenate([top, bot], axis=1).astype(jnp.bfloat16)
    bias = jnp.concatenate([b_iou, b_f], axis=1)
    bias = jnp.pad(bias, ((0, 8 - T), (0, 0)))
    Uf = U_f.astype(jnp.bfloat16)
    type_f = jnp.broadcast_to(type_id.astype(jnp.int32)[:, None], (n, H))

    B = _BLOCK
    grid = (n // B,)
    full = lambda shape: pl.BlockSpec(shape, lambda i: (0,) * len(shape))
    out = pl.pallas_call(
        _tree_cell_kernel,
        grid=grid,
        in_specs=[
            pl.BlockSpec((B, H), lambda i: (i, 0)),
            pl.BlockSpec((B, _K, H), lambda i: (i, 0, 0)),
            pl.BlockSpec((B, _K, H), lambda i: (i, 0, 0)),
            pl.BlockSpec((B, H), lambda i: (i, 0)),
            full((T, 2 * H, 4 * H)),
            full((T, H, H)),
            full((8, 4 * H)),
        ],
        out_specs=[
            pl.BlockSpec((B, H), lambda i: (i, 0)),
            pl.BlockSpec((B, H), lambda i: (i, 0)),
        ],
        out_shape=[
            jax.ShapeDtypeStruct((n, H), x.dtype),
            jax.ShapeDtypeStruct((n, H), x.dtype),
        ],
        compiler_params=pltpu.CompilerParams(
            vmem_limit_bytes=100 * 1024 * 1024),
    )(x, n_h, n_c, type_f, G, Uf, bias)
    return out[0], out[1]


# speedup vs baseline: 1.0527x; 1.0527x over previous
import jax
import jax.numpy as jnp
from jax.experimental import pallas as pl
from jax.experimental.pallas import tpu as pltpu

_K = 32
_H = 128
_N_TYPES = 4
_BLOCK = 200


def _tree_cell_kernel(x_ref, nh_ref, nc_ref, tf_ref,
                      G_ref, Uf_ref, bias_ref,
                      h_ref, c_ref):
    B = x_ref.shape[0]
    x = x_ref[:]
    nh = nh_ref[:]
    nc = nc_ref[:]
    tf = tf_ref[:]

    nhb = nh.astype(jnp.bfloat16)
    h_tilde = jnp.sum(nh, axis=1)
    xh = jnp.concatenate([x, h_tilde], axis=1).astype(jnp.bfloat16)
    nhr = nhb.reshape(B * _K, _H)

    def type_pre(t):
        P = jnp.dot(xh, G_ref[t], preferred_element_type=jnp.float32)
        P = P + bias_ref[t]
        fp = jnp.dot(nhr, Uf_ref[t],
                     preferred_element_type=jnp.float32).reshape(B, _K, _H)
        return P, fp

    P, fp = type_pre(0)
    for t in range(1, _N_TYPES):
        m = tf == t
        P_t, fp_t = type_pre(t)
        P = jnp.where(jnp.concatenate([m, m, m, m], axis=1), P_t, P)
        fp = jnp.where(m[:, None, :], fp_t, fp)
    fp = fp + P[:, 3 * _H:][:, None, :]

    pi = P[:, :_H]
    po = P[:, _H:2 * _H]
    pu = P[:, 2 * _H:3 * _H]
    f_gate = jax.nn.sigmoid(fp)
    c_aggr = jnp.sum(f_gate * nc, axis=1)
    c = jax.nn.sigmoid(pi) * jnp.tanh(pu) + c_aggr
    h = jax.nn.sigmoid(po) * jnp.tanh(c)
    h_ref[:] = h
    c_ref[:] = c


@jax.jit
def kernel(x, n_h, n_c, type_id, W_iou, b_iou, U_iou, W_f, U_f, b_f):
    n = x.shape[0]
    H = _H
    T = _N_TYPES
    top = jnp.concatenate([W_iou, W_f], axis=2)
    bot = jnp.concatenate([U_iou, jnp.zeros((T, H, H), W_iou.dtype)], axis=2)
    G = jnp.concatenate([top, bot], axis=1).astype(jnp.bfloat16)
    bias = jnp.concatenate([b_iou, b_f], axis=1)
    bias = jnp.pad(bias, ((0, 8 - T), (0, 0)))
    Uf = U_f.astype(jnp.bfloat16)
    type_f = jnp.broadcast_to(type_id.astype(jnp.int32)[:, None], (n, H))

    B = _BLOCK
    grid = (n // B,)
    full = lambda shape: pl.BlockSpec(shape, lambda i: (0,) * len(shape))
    out = pl.pallas_call(
        _tree_cell_kernel,
        grid=grid,
        in_specs=[
            pl.BlockSpec((B, H), lambda i: (i, 0)),
            pl.BlockSpec((B, _K, H), lambda i: (i, 0, 0)),
            pl.BlockSpec((B, _K, H), lambda i: (i, 0, 0)),
            pl.BlockSpec((B, H), lambda i: (i, 0)),
            full((T, 2 * H, 4 * H)),
            full((T, H, H)),
            full((8, 4 * H)),
        ],
        out_specs=[
            pl.BlockSpec((B, H), lambda i: (i, 0)),
            pl.BlockSpec((B, H), lambda i: (i, 0)),
        ],
        out_shape=[
            jax.ShapeDtypeStruct((n, H), x.dtype),
            jax.ShapeDtypeStruct((n, H), x.dtype),
        ],
        compiler_params=pltpu.CompilerParams(
            allow_input_fusion=[False, False, False, True,
                                False, False, False]),
    )(x, n_h, n_c, type_f, G, Uf, bias)
    return out[0], out[1]
